# baseline (speedup 1.0000x reference)
import numpy as np
import jax
import jax.numpy as jnp
from jax.experimental import pallas as pl
from jax.experimental.pallas import tpu as pltpu

F32 = jnp.float32

_TAPS = {0: ((0, 3), (1, 1)), 1: ((1, 2), (2, 0))}


def _k1_body(p_ref, w_ref, b_ref, o_ref):
    acc = jnp.dot(p_ref[...], w_ref[...], preferred_element_type=F32)
    o_ref[...] = jnp.maximum(acc + b_ref[...], 0.0)


def _conv1(patches, w1m, b1):
    m = patches.shape[0]
    return pl.pallas_call(
        _k1_body,
        grid=(32,),
        in_specs=[
            pl.BlockSpec((m // 32, 16), lambda i: (i, 0)),
            pl.BlockSpec((16, 128), lambda i: (0, 0)),
            pl.BlockSpec((1, 128), lambda i: (0, 0)),
        ],
        out_specs=pl.BlockSpec((m // 32, 128), lambda i: (i, 0)),
        out_shape=jax.ShapeDtypeStruct((m, 128), F32),
    )(patches, w1m, b1)


def _k2_body(z_ref, w_ref, b_ref, o_ref):
    acc = jnp.zeros((8 * 351, 64), F32)
    for kh in range(4):
        a, t = kh % 2, kh // 2
        for kw in range(4):
            b, s = kw % 2, kw // 2
            blk = z_ref[0, 0, a * 2 + b, t:t + 8, s:s + 351, :]
            acc = acc + jnp.dot(blk.reshape(8 * 351, 128), w_ref[kh * 4 + kw],
                                preferred_element_type=F32)
    o_ref[0, 0] = jnp.maximum(acc + b_ref[...], 0.0).reshape(8, 351, 64)


def _conv2(ph_blocks, w2m, b2):
    return pl.pallas_call(
        _k2_body,
        grid=(8, 2),
        in_specs=[
            pl.BlockSpec((1, 1, 4, 9, 353, 128), lambda n, q: (n, q, 0, 0, 0, 0)),
            pl.BlockSpec((16, 128, 64), lambda n, q: (0, 0, 0)),
            pl.BlockSpec((1, 64), lambda n, q: (0, 0)),
        ],
        out_specs=pl.BlockSpec((1, 1, 8, 351, 64), lambda n, q: (n, q, 0, 0, 0)),
        out_shape=jax.ShapeDtypeStruct((8, 2, 8, 351, 64), F32),
    )(ph_blocks, w2m, b2)


def _k3_body(z_ref, ct_ref, c_ref, o_ref):
    zf = z_ref[...]
    ct = ct_ref[...]
    csq = jnp.sum(ct * ct, axis=0, keepdims=True)
    zsq = jnp.sum(zf * zf, axis=1, keepdims=True)
    dist = (zsq + csq) - 2.0 * jnp.dot(zf, ct, preferred_element_type=F32)
    m = jnp.min(dist, axis=1, keepdims=True)
    iota = jax.lax.broadcasted_iota(jnp.int32, dist.shape, 1)
    idx = jnp.min(jnp.where(dist <= m, iota, 512), axis=1, keepdims=True)
    onehot = (iota == idx).astype(F32)
    o_ref[...] = jnp.dot(onehot, c_ref[...], preferred_element_type=F32)


def _vq(zf, codebook):
    m = zf.shape[0]
    return pl.pallas_call(
        _k3_body,
        grid=(24,),
        in_specs=[
            pl.BlockSpec((m // 24, 64), lambda i: (i, 0)),
            pl.BlockSpec((64, 512), lambda i: (0, 0)),
            pl.BlockSpec((512, 64), lambda i: (0, 0)),
        ],
        out_specs=pl.BlockSpec((m // 24, 64), lambda i: (i, 0)),
        out_shape=jax.ShapeDtypeStruct((m, 64), F32),
    )(zf, codebook.T, codebook)


def _k4_body(z_ref, w_ref, b_ref, o_ref):
    for a in (0, 1):
        for b in (0, 1):
            acc = jnp.zeros((16 * 351, 128), F32)
            for th in (0, 1):
                ro = _TAPS[a][th][0]
                for tw in (0, 1):
                    co = _TAPS[b][tw][0]
                    blk = z_ref[0, ro:ro + 16, co:co + 351, :]
                    acc = acc + jnp.dot(blk.reshape(16 * 351, 64),
                                        w_ref[a * 8 + b * 4 + th * 2 + tw],
                                        preferred_element_type=F32)
            o_ref[0, :, a, :, b, :] = jnp.maximum(acc + b_ref[...], 0.0).reshape(16, 351, 128)


def _convt1(zqp, wt1, b1):
    return pl.pallas_call(
        _k4_body,
        grid=(8,),
        in_specs=[
            pl.BlockSpec((1, 18, 353, 64), lambda n: (n, 0, 0, 0)),
            pl.BlockSpec((16, 64, 128), lambda n: (0, 0, 0)),
            pl.BlockSpec((1, 128), lambda n: (0, 0)),
        ],
        out_specs=pl.BlockSpec((1, 16, 2, 351, 2, 128), lambda n: (n, 0, 0, 0, 0, 0)),
        out_shape=jax.ShapeDtypeStruct((8, 16, 2, 351, 2, 128), F32),
    )(zqp, wt1, b1)


def _k5_body(z_ref, w_ref, b_ref, o_ref):
    for a in (0, 1):
        for b in (0, 1):
            acc = jnp.zeros((8 * 702, 64), F32)
            for th in (0, 1):
                ro = _TAPS[a][th][0]
                for tw in (0, 1):
                    co = _TAPS[b][tw][0]
                    blk = z_ref[0, 0, ro:ro + 8, co:co + 702, :]
                    acc = acc + jnp.dot(blk.reshape(8 * 702, 128),
                                        w_ref[a * 8 + b * 4 + th * 2 + tw],
                                        preferred_element_type=F32)
            o_ref[0, 0, :, a, :, b, :] = jnp.maximum(acc + b_ref[...], 0.0).reshape(8, 702, 64)


def _convt2(d1b, wt2, b2):
    return pl.pallas_call(
        _k5_body,
        grid=(8, 4),
        in_specs=[
            pl.BlockSpec((1, 1, 10, 704, 128), lambda n, q: (n, q, 0, 0, 0)),
            pl.BlockSpec((16, 128, 64), lambda n, q: (0, 0, 0)),
            pl.BlockSpec((1, 64), lambda n, q: (0, 0)),
        ],
        out_specs=pl.BlockSpec((1, 1, 8, 2, 702, 2, 64), lambda n, q: (n, q, 0, 0, 0, 0, 0)),
        out_shape=jax.ShapeDtypeStruct((8, 4, 8, 2, 702, 2, 64), F32),
    )(d1b, wt2, b2)


def _k6_body(h_ref, v_ref, b_ref, o_ref):
    acc = jnp.zeros((16, 1407, 64), F32)
    for kh in range(3):
        for kw in range(3):
            acc = acc + h_ref[0, 0, kh:kh + 16, kw:kw + 1407, :] * v_ref[kh * 3 + kw][None, None, :]
    out = jnp.sum(acc, axis=-1) + b_ref[...]
    o_ref[0, 0] = jnp.tanh(jnp.maximum(out, 0.0))


def _outconv(hb, v, ob):
    return pl.pallas_call(
        _k6_body,
        grid=(8, 4),
        in_specs=[
            pl.BlockSpec((1, 1, 18, 1409, 64), lambda n, q: (n, q, 0, 0, 0)),
            pl.BlockSpec((16, 64), lambda n, q: (0, 0)),
            pl.BlockSpec((1, 1), lambda n, q: (0, 0)),
        ],
        out_specs=pl.BlockSpec((1, 1, 16, 1407), lambda n, q: (n, q, 0, 0)),
        out_shape=jax.ShapeDtypeStruct((8, 4, 16, 1407), F32),
    )(hb, v, ob)


def kernel(x, enc_w1, enc_b1, enc_w2, enc_b2, codebook,
           dec_w1, dec_b1, dec_w2, dec_b2, out_w, out_b):
    xp = jnp.pad(x[:, 0], ((0, 0), (1, 1), (1, 1)))
    patches = jnp.stack(
        [xp[:, kh:kh + 64:2, kw:kw + 1406:2] for kh in range(4) for kw in range(4)],
        axis=-1).reshape(8 * 32 * 703, 16)
    w1m = enc_w1.reshape(128, 16).T
    z = _conv1(patches, w1m, enc_b1.reshape(1, 128)).reshape(8, 32, 703, 128)

    zp = jnp.pad(z, ((0, 0), (1, 1), (1, 1), (0, 0)))
    phases = []
    for a in (0, 1):
        for b in (0, 1):
            p = zp[:, a::2, b::2, :]
            if p.shape[2] == 352:
                p = jnp.pad(p, ((0, 0), (0, 0), (0, 1), (0, 0)))
            phases.append(p)
    ph = jnp.stack(phases, 1)
    ph_blocks = jnp.stack([ph[:, :, 8 * q:8 * q + 9] for q in range(2)], 1)
    w2m = jnp.stack([enc_w2[:, :, kh, kw].T for kh in range(4) for kw in range(4)])
    ze = _conv2(ph_blocks, w2m, enc_b2.reshape(1, 64)).reshape(8 * 16 * 351, 64)

    zq = _vq(ze, codebook).reshape(8, 16, 351, 64)

    zqp = jnp.pad(zq, ((0, 0), (1, 1), (1, 1), (0, 0)))
    wt1 = jnp.stack([dec_w1[:, :, _TAPS[a][th][1], _TAPS[b][tw][1]]
                     for a in (0, 1) for b in (0, 1) for th in (0, 1) for tw in (0, 1)])
    d1 = _convt1(zqp, wt1, dec_b1.reshape(1, 128)).reshape(8, 32, 702, 128)

    d1p = jnp.pad(d1, ((0, 0), (1, 1), (1, 1), (0, 0)))
    d1b = jnp.stack([d1p[:, 8 * q:8 * q + 10] for q in range(4)], 1)
    wt2 = jnp.stack([dec_w2[:, :, _TAPS[a][th][1], _TAPS[b][tw][1]]
                     for a in (0, 1) for b in (0, 1) for th in (0, 1) for tw in (0, 1)])
    d2 = _convt2(d1b, wt2, dec_b2.reshape(1, 64)).reshape(8, 64, 1404, 64)

    d2p = jnp.pad(d2, ((0, 0), (1, 1), (1, 1), (0, 0)))
    iw = np.concatenate([[0], (np.arange(1407) * 1404) // 1407 + 1, [1405]])
    hb = jnp.take(jnp.stack([d2p[:, 16 * q:16 * q + 18] for q in range(4)], 1),
                  iw, axis=3)
    v = jnp.pad(out_w[0].reshape(64, 9).T, ((0, 7), (0, 0)))
    out = _outconv(hb, v, out_b.reshape(1, 1))
    return out.reshape(8, 1, 64, 1407)
